# baseline (speedup 1.0000x reference)
import functools

import jax
import jax.numpy as jnp
from jax import lax
from jax.experimental import pallas as pl
from jax.experimental.pallas import tpu as pltpu
from jax.experimental.pallas import tpu_sc as plsc

_LANES = 16
_NW = 32
_CHUNK = 128
_FIRE = 8


def _sc_gather(table, idx2):
    n_chunks = idx2.shape[0]
    ch_per_w = n_chunks // _NW
    rows_per_w = ch_per_w * _CHUNK
    lines_per_w = rows_per_w // 8
    lines_per_ch = _CHUNK // 8
    mesh = plsc.VectorSubcoreMesh(core_axis_name="c", subcore_axis_name="s")

    @functools.partial(
        pl.kernel,
        mesh=mesh,
        out_type=jax.ShapeDtypeStruct(
            (n_chunks * _CHUNK // 8, 8 * _LANES), jnp.float32
        ),
        scratch_types=[
            pltpu.VMEM((ch_per_w, _CHUNK), jnp.int32),
            pltpu.VMEM((2, _CHUNK, 8 * _LANES), jnp.float32),
            pltpu.VMEM((lines_per_w, 8 * _LANES), jnp.float32),
            pltpu.SemaphoreType.DMA,
        ],
    )
    def k(table_hbm, idx_hbm, out_hbm, idx_v, g_v, lines_v, sem):
        wid = lax.axis_index("s") * 2 + lax.axis_index("c")
        cbase = wid * ch_per_w
        pltpu.sync_copy(idx_hbm.at[pl.ds(cbase, ch_per_w)], idx_v)

        pltpu.async_copy(table_hbm.at[idx_v.at[0]], g_v.at[0], sem)

        def body(j, carry):
            @pl.when(j + 1 < ch_per_w)
            def _():
                pltpu.async_copy(
                    table_hbm.at[idx_v.at[j + 1]], g_v.at[(j + 1) % 2], sem
                )

            pltpu.make_async_copy(
                table_hbm.at[idx_v.at[j]], g_v.at[j % 2], sem
            ).wait()

            def pack(r8, c2):
                for k2 in range(8):
                    v = g_v[j % 2, r8 * 8 + k2, pl.ds(0, _LANES)]
                    lines_v[j * lines_per_ch + r8,
                            pl.ds(k2 * _LANES, _LANES)] = v
                return c2

            lax.fori_loop(0, lines_per_ch, pack, 0)
            return carry

        lax.fori_loop(0, ch_per_w, body, 0)
        pltpu.sync_copy(
            lines_v, out_hbm.at[pl.ds(wid * lines_per_w, lines_per_w)]
        )

    return k(table, idx2)


def _tc_body(b_dim, p_blk, m_dim, cin, cout, w_ref, xg_ref, bias_ref, out_ref):
    zw = cout * cin
    w = w_ref[...]
    xg = xg_ref[...]
    zmod = lax.broadcasted_iota(jnp.int32, (1, 1, zw), 2) % cin
    psums = []
    for b in range(b_dim):
        cs = [
            jnp.broadcast_to(
                xg[:, :, b * cin + i][:, :, None], (p_blk, m_dim, zw)
            )
            for i in range(cin)
        ]
        xge = cs[-1]
        for i in range(cin - 2, -1, -1):
            xge = jnp.where(zmod == i, cs[i], xge)
        psums.append(w * xge)
    psa = jnp.concatenate([p[None] for p in psums], axis=0)
    psa = psa.reshape(b_dim * p_blk * m_dim, zw)
    zi = lax.broadcasted_iota(jnp.int32, (zw, cout), 0)
    oi = lax.broadcasted_iota(jnp.int32, (zw, cout), 1)
    sel = (zi // cin == oi).astype(jnp.float32)
    y = lax.dot(psa, sel, preferred_element_type=jnp.float32)
    y = jnp.sum(y.reshape(b_dim, p_blk, m_dim, cout), axis=2)
    y = y + bias_ref[...][None]
    y = jnp.where(y > 0, y, jnp.exp(jnp.minimum(y, 0.0)) - 1.0)
    out_ref[...] = y


def kernel(x_batch, neighbor_id_lstlst, weights, bias):
    b_dim, n, cin = x_batch.shape
    m_dim = neighbor_id_lstlst.shape[1]
    cout = weights.shape[2]

    xt = jnp.transpose(x_batch, (1, 0, 2)).reshape(n, b_dim * cin)
    xt = jnp.concatenate(
        [xt, jnp.zeros((n, 8 * _LANES - b_dim * cin), xt.dtype)], axis=1
    )
    idx_flat = neighbor_id_lstlst.reshape(-1)
    n_rows = idx_flat.shape[0]
    rows_pad = -n_rows % (_NW * _FIRE * _CHUNK)
    idx2 = jnp.concatenate(
        [idx_flat, jnp.zeros((rows_pad,), idx_flat.dtype)]
    ).reshape(-1, _CHUNK)
    xg = _sc_gather(xt, idx2)
    xg = xg[: n_rows // 8].reshape(n, m_dim, _LANES)

    w_r = weights.reshape(n, m_dim, cout * cin)

    p_blk = 80
    grid = (n // p_blk,)
    out = pl.pallas_call(
        functools.partial(_tc_body, b_dim, p_blk, m_dim, cin, cout),
        grid=grid,
        in_specs=[
            pl.BlockSpec((p_blk, m_dim, cout * cin), lambda i: (i, 0, 0)),
            pl.BlockSpec((p_blk, m_dim, _LANES), lambda i: (i, 0, 0)),
            pl.BlockSpec((1, cout), lambda i: (0, 0)),
        ],
        out_specs=pl.BlockSpec((b_dim, p_blk, cout), lambda i: (0, i, 0)),
        out_shape=jax.ShapeDtypeStruct((b_dim, n, cout), jnp.float32),
        compiler_params=pltpu.CompilerParams(
            dimension_semantics=("arbitrary",),
        ),
    )(w_r, xg, bias.reshape(1, cout))
    return out


# speedup vs baseline: 1.0497x; 1.0497x over previous
import functools

import jax
import jax.numpy as jnp
from jax import lax
from jax.experimental import pallas as pl
from jax.experimental.pallas import tpu as pltpu
from jax.experimental.pallas import tpu_sc as plsc

_LANES = 16
_NW = 32
_CHUNK = 128
_FIRE = 8


def _sc_gather(table, idx2):
    n_chunks = idx2.shape[0]
    ch_per_w = n_chunks // _NW
    rows_per_w = ch_per_w * _CHUNK
    mesh = plsc.VectorSubcoreMesh(core_axis_name="c", subcore_axis_name="s")

    @functools.partial(
        pl.kernel,
        mesh=mesh,
        out_type=jax.ShapeDtypeStruct(
            (n_chunks * _CHUNK, 8 * _LANES), jnp.float32
        ),
        scratch_types=[
            pltpu.VMEM((ch_per_w, _CHUNK), jnp.int32),
            pltpu.VMEM((2, _CHUNK, 8 * _LANES), jnp.float32),
            pltpu.SemaphoreType.DMA,
            pltpu.SemaphoreType.DMA,
        ],
    )
    def k(table_hbm, idx_hbm, out_hbm, idx_v, g_v, sem_in, sem_out):
        wid = lax.axis_index("s") * 2 + lax.axis_index("c")
        cbase = wid * ch_per_w
        rbase = wid * rows_per_w
        pltpu.sync_copy(idx_hbm.at[pl.ds(cbase, ch_per_w)], idx_v)

        pltpu.async_copy(table_hbm.at[idx_v.at[0]], g_v.at[0], sem_in)

        def body(j, carry):
            @pl.when(j >= 1)
            def _():
                pltpu.make_async_copy(
                    g_v.at[(j + 1) % 2],
                    out_hbm.at[pl.ds(rbase + (j - 1) * _CHUNK, _CHUNK)],
                    sem_out,
                ).wait()

            @pl.when(j + 1 < ch_per_w)
            def _():
                pltpu.async_copy(
                    table_hbm.at[idx_v.at[j + 1]], g_v.at[(j + 1) % 2],
                    sem_in,
                )

            pltpu.make_async_copy(
                table_hbm.at[idx_v.at[j]], g_v.at[j % 2], sem_in
            ).wait()
            pltpu.async_copy(
                g_v.at[j % 2],
                out_hbm.at[pl.ds(rbase + j * _CHUNK, _CHUNK)],
                sem_out,
            )
            return carry

        lax.fori_loop(0, ch_per_w, body, 0)
        pltpu.make_async_copy(
            g_v.at[(ch_per_w - 1) % 2],
            out_hbm.at[pl.ds(rbase + (ch_per_w - 1) * _CHUNK, _CHUNK)],
            sem_out,
        ).wait()

    return k(table, idx2)


def _tc_body(b_dim, p_blk, m_dim, cin, cout, w_ref, xg_ref, bias_ref, out_ref):
    zw = cout * cin
    w = w_ref[...]
    xg = jnp.reshape(xg_ref[...], (p_blk, m_dim, 8 * _LANES))
    zmod = lax.broadcasted_iota(jnp.int32, (1, 1, zw), 2) % cin
    psums = []
    for b in range(b_dim):
        cs = [
            jnp.broadcast_to(
                xg[:, :, b * cin + i][:, :, None], (p_blk, m_dim, zw)
            )
            for i in range(cin)
        ]
        xge = cs[-1]
        for i in range(cin - 2, -1, -1):
            xge = jnp.where(zmod == i, cs[i], xge)
        psums.append(w * xge)
    psa = jnp.concatenate([p[None] for p in psums], axis=0)
    psa = psa.reshape(b_dim * p_blk * m_dim, zw)
    zi = lax.broadcasted_iota(jnp.int32, (zw, cout), 0)
    oi = lax.broadcasted_iota(jnp.int32, (zw, cout), 1)
    sel = (zi // cin == oi).astype(jnp.float32)
    y = lax.dot(psa, sel, preferred_element_type=jnp.float32)
    y = jnp.sum(y.reshape(b_dim, p_blk, m_dim, cout), axis=2)
    y = y + bias_ref[...][None]
    y = jnp.where(y > 0, y, jnp.exp(jnp.minimum(y, 0.0)) - 1.0)
    out_ref[...] = y


def kernel(x_batch, neighbor_id_lstlst, weights, bias):
    b_dim, n, cin = x_batch.shape
    m_dim = neighbor_id_lstlst.shape[1]
    cout = weights.shape[2]

    xt = jnp.transpose(x_batch, (1, 0, 2)).reshape(n, b_dim * cin)
    xt = jnp.concatenate(
        [xt, jnp.zeros((n, 8 * _LANES - b_dim * cin), xt.dtype)], axis=1
    )
    idx_flat = neighbor_id_lstlst.reshape(-1)
    n_rows = idx_flat.shape[0]
    rows_pad = -n_rows % (_NW * _FIRE * _CHUNK)
    idx2 = jnp.concatenate(
        [idx_flat, jnp.zeros((rows_pad,), idx_flat.dtype)]
    ).reshape(-1, _CHUNK)
    xg = _sc_gather(xt, idx2)

    w_r = weights.reshape(n, m_dim, cout * cin)

    p_blk = 80
    grid = (n // p_blk,)
    out = pl.pallas_call(
        functools.partial(_tc_body, b_dim, p_blk, m_dim, cin, cout),
        grid=grid,
        in_specs=[
            pl.BlockSpec((p_blk, m_dim, cout * cin), lambda i: (i, 0, 0)),
            pl.BlockSpec((p_blk * m_dim, 8 * _LANES), lambda i: (i, 0)),
            pl.BlockSpec((1, cout), lambda i: (0, 0)),
        ],
        out_specs=pl.BlockSpec((b_dim, p_blk, cout), lambda i: (0, i, 0)),
        out_shape=jax.ShapeDtypeStruct((b_dim, n, cout), jnp.float32),
        compiler_params=pltpu.CompilerParams(
            dimension_semantics=("arbitrary",),
        ),
    )(w_r, xg, bias.reshape(1, cout))
    return out


# speedup vs baseline: 1.2194x; 1.1616x over previous
import functools

import jax
import jax.numpy as jnp
from jax import lax
from jax.experimental import pallas as pl
from jax.experimental.pallas import tpu as pltpu
from jax.experimental.pallas import tpu_sc as plsc

_LANES = 16
_NW = 32
_CHUNK = 128
_FIRE = 8


_NQ = 5


def _sc_gather(table, idx2):
    n_chunks = idx2.shape[0]
    ch_per_w = n_chunks // _NW
    rows_per_w = ch_per_w * _CHUNK
    lines_per_w = rows_per_w // 8
    lines_per_q = lines_per_w // _NQ
    rows_per_q = rows_per_w // _NQ
    mesh = plsc.VectorSubcoreMesh(core_axis_name="c", subcore_axis_name="s")

    @functools.partial(
        pl.kernel,
        mesh=mesh,
        out_type=jax.ShapeDtypeStruct(
            (n_chunks * _CHUNK // 8, 8 * _LANES), jnp.float32
        ),
        scratch_types=[
            pltpu.VMEM((ch_per_w, _CHUNK), jnp.int32),
            pltpu.VMEM((rows_per_w, _LANES), jnp.float32),
            pltpu.VMEM((lines_per_q, 8 * _LANES), jnp.float32),
            pltpu.SemaphoreType.DMA,
        ],
        compiler_params=pltpu.CompilerParams(use_tc_tiling_on_sc=False),
    )
    def k(table_hbm, idx_hbm, out_hbm, idx_v, rows_v, buf_v, sem):
        wid = lax.axis_index("s") * 2 + lax.axis_index("c")
        cbase = wid * ch_per_w
        pltpu.sync_copy(idx_hbm.at[pl.ds(cbase, ch_per_w)], idx_v)

        def body(jo, carry):
            copies = []
            for k2 in range(_FIRE):
                j = jo * _FIRE + k2
                copies.append(
                    pltpu.async_copy(
                        table_hbm.at[idx_v.at[j]],
                        rows_v.at[pl.ds(j * _CHUNK, _CHUNK)],
                        sem,
                    )
                )
            for c in copies:
                c.wait()
            return carry

        lax.fori_loop(0, ch_per_w // _FIRE, body, 0)

        for q in range(_NQ):
            def pack(r8, carry):
                for k2 in range(8):
                    v = rows_v[q * rows_per_q + r8 * 8 + k2, :]
                    buf_v[r8, pl.ds(k2 * _LANES, _LANES)] = v
                return carry

            lax.fori_loop(0, lines_per_q, pack, 0)
            pltpu.sync_copy(
                buf_v,
                out_hbm.at[pl.ds(wid * lines_per_w + q * lines_per_q,
                                 lines_per_q)],
            )

    return k(table, idx2)


def _tc_body(b_dim, p_blk, m_dim, cin, cout, w_ref, xg_ref, bias_ref, out_ref):
    zw = cout * cin
    w = w_ref[...]
    xg = xg_ref[...]
    zmod = lax.broadcasted_iota(jnp.int32, (1, 1, zw), 2) % cin
    psums = []
    for b in range(b_dim):
        cs = [
            jnp.broadcast_to(
                xg[:, :, b * cin + i][:, :, None], (p_blk, m_dim, zw)
            )
            for i in range(cin)
        ]
        xge = cs[-1]
        for i in range(cin - 2, -1, -1):
            xge = jnp.where(zmod == i, cs[i], xge)
        psums.append(w * xge)
    psa = jnp.concatenate([p[None] for p in psums], axis=0)
    psa = psa.reshape(b_dim * p_blk * m_dim, zw)
    zi = lax.broadcasted_iota(jnp.int32, (zw, cout), 0)
    oi = lax.broadcasted_iota(jnp.int32, (zw, cout), 1)
    sel = (zi // cin == oi).astype(jnp.float32)
    y = lax.dot(psa, sel, preferred_element_type=jnp.float32)
    y = jnp.sum(y.reshape(b_dim, p_blk, m_dim, cout), axis=2)
    y = y + bias_ref[...][None]
    y = jnp.where(y > 0, y, jnp.exp(jnp.minimum(y, 0.0)) - 1.0)
    out_ref[...] = y


def kernel(x_batch, neighbor_id_lstlst, weights, bias):
    b_dim, n, cin = x_batch.shape
    m_dim = neighbor_id_lstlst.shape[1]
    cout = weights.shape[2]

    xt = jnp.transpose(x_batch, (1, 0, 2)).reshape(n, b_dim * cin)
    xt = jnp.concatenate(
        [xt, jnp.zeros((n, _LANES - b_dim * cin), xt.dtype)], axis=1
    )
    idx_flat = neighbor_id_lstlst.reshape(-1)
    n_rows = idx_flat.shape[0]
    rows_pad = -n_rows % (_NW * _FIRE * _CHUNK)
    idx2 = jnp.concatenate(
        [idx_flat, jnp.zeros((rows_pad,), idx_flat.dtype)]
    ).reshape(-1, _CHUNK)
    xg = _sc_gather(xt, idx2)
    xg = xg[: n_rows // 8].reshape(n, m_dim, _LANES)

    w_r = weights.reshape(n, m_dim, cout * cin)

    p_blk = 80
    grid = (n // p_blk,)
    out = pl.pallas_call(
        functools.partial(_tc_body, b_dim, p_blk, m_dim, cin, cout),
        grid=grid,
        in_specs=[
            pl.BlockSpec((p_blk, m_dim, cout * cin), lambda i: (i, 0, 0)),
            pl.BlockSpec((p_blk, m_dim, _LANES), lambda i: (i, 0, 0)),
            pl.BlockSpec((1, cout), lambda i: (0, 0)),
        ],
        out_specs=pl.BlockSpec((b_dim, p_blk, cout), lambda i: (0, i, 0)),
        out_shape=jax.ShapeDtypeStruct((b_dim, n, cout), jnp.float32),
        compiler_params=pltpu.CompilerParams(
            dimension_semantics=("arbitrary",),
        ),
    )(w_r, xg, bias.reshape(1, cout))
    return out
